# Initial kernel scaffold; baseline (speedup 1.0000x reference)
import jax
import jax.numpy as jnp
from jax.experimental import pallas as pl


def kernel(bins, target_depth_maps):
    raise NotImplementedError("write your pallas kernel here")


# speedup vs baseline: 2.3333x; 2.3333x over previous
import functools

import jax
import jax.numpy as jnp
from jax.experimental import pallas as pl
from jax.experimental.pallas import tpu as pltpu
from jax.experimental.pallas import tpu_sc as plsc

_L = 16
_NT = 16
_NC = 2


def _sc_body(bins_hbm, target_hbm, out_hbm, bins_v, cent_v, sortc_v,
             minacc_v, maxacc_v, redmin_v, redmax_v, targ_v, sa_v, pb_v,
             gmin_v, gmax_v, gcnt_v, rank_st, val_st, cnt_st, rankall_v,
             valall_v, out_st, sh_rank, sh_val, sh_min, sh_max, sh_cnt):
    E = cent_v.shape[0]
    B = bins_hbm.shape[0] // E
    TPT = targ_v.shape[0]
    T = target_hbm.shape[0] // B
    B2 = B // _NC
    KCH = E // _L
    cid = jax.lax.axis_index("c")
    sid = jax.lax.axis_index("s")
    lane = jax.lax.iota(jnp.int32, _L)
    INF = jnp.float32(jnp.inf)

    steps = []
    s = E // 2
    while s >= 1:
        steps.append(s)
        s //= 2

    for lb in range(B2):
        b = cid * B2 + lb

        pltpu.sync_copy(bins_hbm.at[pl.ds(b * E, E)], bins_v.at[pl.ds(0, E)])
        for k in range(KCH):
            e0 = bins_v[pl.ds(k * _L, _L)]
            e1 = plsc.load_gather(bins_v, [lane + (k * _L + 1)])
            c = 0.5 * (e0 + e1)
            if k == KCH - 1:
                c = jnp.where(lane == _L - 1, INF, c)
            cent_v[pl.ds(k * _L, _L)] = c

        cvec = cent_v[pl.ds(sid * _L, _L)]
        pvec = sid * _L + lane

        def rank_body(k, r):
            vq = cent_v[pl.ds(k * _L, _L)]
            for qi in range(_L):
                cq = vq[qi]
                q = k * _L + qi
                lt = cq < cvec
                eq = (cq == cvec) & (q < pvec)
                r = r + jnp.where(lt, 1, 0) + jnp.where(eq, 1, 0)
            return r

        rank = jax.lax.fori_loop(0, KCH, rank_body,
                                 jnp.zeros((_L,), jnp.int32))
        rank_st[...] = rank
        val_st[...] = cvec
        pltpu.sync_copy(rank_st, sh_rank.at[lb, sid])
        pltpu.sync_copy(val_st, sh_val.at[lb, sid])
        plsc.subcore_barrier()
        pltpu.sync_copy(sh_rank.at[lb], rankall_v)
        pltpu.sync_copy(sh_val.at[lb], valall_v)
        for k in range(_NT):
            plsc.store_scatter(sortc_v, [rankall_v[k] + lb * E],
                               valall_v[k])

        def init_body(k, _):
            minacc_v[pl.ds(k * _L, _L)] = jnp.full((_L,), INF, jnp.float32)
            maxacc_v[pl.ds(k * _L, _L)] = jnp.full((_L,), -INF, jnp.float32)
            return 0

        jax.lax.fori_loop(0, (_L * E) // _L, init_body, 0)

        pltpu.sync_copy(target_hbm.at[pl.ds(b * T + sid * TPT, TPT)], targ_v)
        lane_base = lane * E

        def tgt_body(i, cnt):
            t = targ_v[pl.ds(i * _L, _L)]
            valid = t >= 0.001
            j = jnp.zeros((_L,), jnp.int32)
            for st in steps:
                probe = j + (st - 1 + lb * E)
                v = plsc.load_gather(sortc_v, [probe])
                j = j + jnp.where(v <= t, st, 0)
            addr = lane_base + j
            tmin = jnp.where(valid, t, INF)
            tmax = jnp.where(valid, t, -INF)
            cur = plsc.load_gather(minacc_v, [addr])
            plsc.store_scatter(minacc_v, [addr], jnp.minimum(cur, tmin))
            cur2 = plsc.load_gather(maxacc_v, [addr])
            plsc.store_scatter(maxacc_v, [addr], jnp.maximum(cur2, tmax))
            return cnt + jnp.where(valid, 1.0, 0.0)

        cnt = jax.lax.fori_loop(0, TPT // _L, tgt_body,
                                jnp.zeros((_L,), jnp.float32))

        def lr_body(k, _):
            base = k * _L
            mn = minacc_v[pl.ds(base, _L)]
            mx = maxacc_v[pl.ds(base, _L)]
            for l in range(1, _L):
                mn = jnp.minimum(mn, minacc_v[pl.ds(l * E + base, _L)])
                mx = jnp.maximum(mx, maxacc_v[pl.ds(l * E + base, _L)])
            redmin_v[pl.ds(base, _L)] = mn
            redmax_v[pl.ds(base, _L)] = mx
            return 0

        jax.lax.fori_loop(0, KCH, lr_body, 0)
        pltpu.sync_copy(redmin_v, sh_min.at[lb, sid])
        pltpu.sync_copy(redmax_v, sh_max.at[lb, sid])
        cnt_st[...] = cnt
        pltpu.sync_copy(cnt_st, sh_cnt.at[lb, sid])

    plsc.subcore_barrier()

    @pl.when(sid < B2)
    def _():
        lb = sid
        b = cid * B2 + lb
        pltpu.sync_copy(sh_min.at[lb], gmin_v)
        pltpu.sync_copy(sh_max.at[lb], gmax_v)
        pltpu.sync_copy(sh_cnt.at[lb], gcnt_v)

        def comb_body(k, _):
            base = k * _L
            mn = gmin_v[0, pl.ds(base, _L)]
            mx = gmax_v[0, pl.ds(base, _L)]
            for tt in range(1, _NT):
                mn = jnp.minimum(mn, gmin_v[tt, pl.ds(base, _L)])
                mx = jnp.maximum(mx, gmax_v[tt, pl.ds(base, _L)])
            redmin_v[pl.ds(base, _L)] = mn
            redmax_v[pl.ds(base, _L)] = mx
            return 0

        jax.lax.fori_loop(0, KCH, comb_body, 0)

        sa_v[pl.ds(E, _L)] = jnp.full((_L,), INF, jnp.float32)
        carry = -INF
        for k in range(KCH - 1, -1, -1):
            v = redmin_v[pl.ds(k * _L, _L)]
            rnm = jax.lax.rev(-v, (0,))
            sm = jnp.maximum(plsc.cummax(rnm), carry)
            sa_v[pl.ds(k * _L, _L)] = jax.lax.rev(-sm, (0,))
            carry = jnp.max(sm)

        carry2 = -INF
        for k in range(KCH):
            v = redmax_v[pl.ds(k * _L, _L)]
            pm = jnp.maximum(plsc.cummax(v), carry2)
            pb_v[pl.ds(k * _L, _L)] = pm
            carry2 = jnp.max(pm)

        s0 = jnp.zeros((_L,), jnp.float32)
        s1 = jnp.zeros((_L,), jnp.float32)
        for k in range(KCH):
            scv = plsc.load_gather(sortc_v, [lane + (lb * E + k * _L)])
            sa1 = plsc.load_gather(sa_v, [lane + (k * _L + 1)])
            pb = pb_v[pl.ds(k * _L, _L)]
            res = jnp.minimum(sa1 - scv, scv - pb)
            resp = jnp.minimum(res, jnp.abs(scv))
            real = (k * _L + lane) <= (E - 2)
            s0 = s0 + jnp.where(real, res, 0.0)
            s1 = s1 + jnp.where(real, resp, 0.0)

        cacc = gcnt_v[0]
        for tt in range(1, _NT):
            cacc = cacc + gcnt_v[tt]

        t0 = jnp.sum(s0)
        t1 = jnp.sum(s1)
        tc = jnp.sum(cacc)
        outv = jnp.where(lane == 0, t0,
                         jnp.where(lane == 1, t1,
                                   jnp.where(lane == 2, tc, 0.0)))
        out_st[...] = outv
        pltpu.sync_copy(out_st, out_hbm.at[pl.ds(b * _L, _L)])


def _scratch_types(B, E, T):
    f32 = jnp.float32
    i32 = jnp.int32
    TPT = T // _NT
    return [
        pltpu.VMEM((E + 128,), f32),
        pltpu.VMEM((E,), f32),
        pltpu.VMEM(((B // _NC) * E,), f32),
        pltpu.VMEM((_L * E,), f32),
        pltpu.VMEM((_L * E,), f32),
        pltpu.VMEM((E,), f32),
        pltpu.VMEM((E,), f32),
        pltpu.VMEM((TPT,), f32),
        pltpu.VMEM((E + 128,), f32),
        pltpu.VMEM((E,), f32),
        pltpu.VMEM((_NT, E), f32),
        pltpu.VMEM((_NT, E), f32),
        pltpu.VMEM((_NT, _L), f32),
        pltpu.VMEM((_L,), i32),
        pltpu.VMEM((_L,), f32),
        pltpu.VMEM((_L,), f32),
        pltpu.VMEM((_NT, _L), i32),
        pltpu.VMEM((_NT, _L), f32),
        pltpu.VMEM((_L,), f32),
        pltpu.VMEM_SHARED((B // _NC, _NT, _L), i32),
        pltpu.VMEM_SHARED((B // _NC, _NT, _L), f32),
        pltpu.VMEM_SHARED((B // _NC, _NT, E), f32),
        pltpu.VMEM_SHARED((B // _NC, _NT, E), f32),
        pltpu.VMEM_SHARED((B // _NC, _NT, _L), f32),
    ]


def _make_call(B, E, T):
    mesh = plsc.VectorSubcoreMesh(core_axis_name="c", subcore_axis_name="s",
                                  num_cores=_NC, num_subcores=_NT)
    return functools.partial(
        pl.kernel, mesh=mesh,
        out_type=jax.ShapeDtypeStruct((B * _L,), jnp.float32),
        scratch_types=_scratch_types(B, E, T),
        compiler_params=pltpu.CompilerParams(use_tc_tiling_on_sc=False,
                                             needs_layout_passes=False),
    )(_sc_body)


_sc_call = _make_call(4, 256, 50176)


def kernel(bins, target_depth_maps):
    B = bins.shape[0]
    target = target_depth_maps.reshape(-1).astype(jnp.float32)
    out = _sc_call(bins.reshape(-1).astype(jnp.float32), target).reshape(B, _L)
    s0, s1, cnt = out[:, 0], out[:, 1], out[:, 2]
    max_len = jnp.max(cnt)
    return jnp.sum(jnp.where(cnt < max_len, s1, s0))


# speedup vs baseline: 3.1123x; 1.3338x over previous
import functools

import jax
import jax.numpy as jnp
from jax.experimental import pallas as pl
from jax.experimental.pallas import tpu as pltpu
from jax.experimental.pallas import tpu_sc as plsc

_L = 16
_NT = 16
_NC = 2


def _sc_body(bins_hbm, target_hbm, out_hbm, bins_v, cent_v, sortc_v,
             minacc_v, maxacc_v, redmin_v, redmax_v, targ_v, sa_v, pb_v,
             gmin_v, gmax_v, gcnt_v, rank_st, val_st, cnt_st, rankall_v,
             valall_v, out_st, sh_rank, sh_val, sh_min, sh_max, sh_cnt):
    E = cent_v.shape[0]
    B = bins_hbm.shape[0] // E
    TPT = targ_v.shape[0]
    T = target_hbm.shape[0] // B
    B2 = B // _NC
    KCH = E // _L
    cid = jax.lax.axis_index("c")
    sid = jax.lax.axis_index("s")
    lane = jax.lax.iota(jnp.int32, _L)
    INF = jnp.float32(jnp.inf)

    steps = []
    s = E // 2
    while s >= 1:
        steps.append(s)
        s //= 2

    for lb in range(B2):
        b = cid * B2 + lb

        pltpu.sync_copy(bins_hbm.at[pl.ds(b * E, E)], bins_v.at[pl.ds(0, E)])
        for k in range(KCH):
            e0 = bins_v[pl.ds(k * _L, _L)]
            e1 = plsc.load_gather(bins_v, [lane + (k * _L + 1)])
            c = 0.5 * (e0 + e1)
            if k == KCH - 1:
                c = jnp.where(lane == _L - 1, INF, c)
            cent_v[pl.ds(k * _L, _L)] = c

        cvec = cent_v[pl.ds(sid * _L, _L)]
        pvec = sid * _L + lane

        def rank_body(k, r):
            vq = cent_v[pl.ds(k * _L, _L)]
            for qi in range(_L):
                cq = vq[qi]
                q = k * _L + qi
                lt = cq < cvec
                eq = (cq == cvec) & (q < pvec)
                r = r + jnp.where(lt, 1, 0) + jnp.where(eq, 1, 0)
            return r

        rank = jax.lax.fori_loop(0, KCH, rank_body,
                                 jnp.zeros((_L,), jnp.int32))
        rank_st[...] = rank
        val_st[...] = cvec
        pltpu.sync_copy(rank_st, sh_rank.at[lb, sid])
        pltpu.sync_copy(val_st, sh_val.at[lb, sid])
        plsc.subcore_barrier()
        pltpu.sync_copy(sh_rank.at[lb], rankall_v)
        pltpu.sync_copy(sh_val.at[lb], valall_v)
        for k in range(_NT):
            plsc.store_scatter(sortc_v, [rankall_v[k] + lb * E],
                               valall_v[k])

        def init_body(k, _):
            minacc_v[pl.ds(k * _L, _L)] = jnp.full((_L,), INF, jnp.float32)
            maxacc_v[pl.ds(k * _L, _L)] = jnp.full((_L,), -INF, jnp.float32)
            return 0

        jax.lax.fori_loop(0, (_L * E) // _L, init_body, 0)

        pltpu.sync_copy(target_hbm.at[pl.ds(b * T + sid * TPT, TPT)], targ_v)
        lane_base = lane * E

        U = 4

        def tgt_body(i, cnt):
            ts = [targ_v[pl.ds((i * U + u) * _L, _L)] for u in range(U)]
            js = []
            for u in range(U):
                t = ts[u]
                j = jnp.zeros((_L,), jnp.int32)
                for st in steps:
                    probe = j + (st - 1 + lb * E)
                    v = plsc.load_gather(sortc_v, [probe])
                    j = j + jnp.where(v <= t, st, 0)
                js.append(j)
            for u in range(U):
                t = ts[u]
                valid = t >= 0.001
                addr = lane_base + js[u]
                tmin = jnp.where(valid, t, INF)
                tmax = jnp.where(valid, t, -INF)
                cur = plsc.load_gather(minacc_v, [addr])
                plsc.store_scatter(minacc_v, [addr],
                                   jnp.minimum(cur, tmin))
                cur2 = plsc.load_gather(maxacc_v, [addr])
                plsc.store_scatter(maxacc_v, [addr],
                                   jnp.maximum(cur2, tmax))
                cnt = cnt + jnp.where(valid, 1.0, 0.0)
            return cnt

        cnt = jax.lax.fori_loop(0, TPT // (_L * U), tgt_body,
                                jnp.zeros((_L,), jnp.float32))

        def lr_body(k, _):
            base = k * _L
            mn = minacc_v[pl.ds(base, _L)]
            mx = maxacc_v[pl.ds(base, _L)]
            for l in range(1, _L):
                mn = jnp.minimum(mn, minacc_v[pl.ds(l * E + base, _L)])
                mx = jnp.maximum(mx, maxacc_v[pl.ds(l * E + base, _L)])
            redmin_v[pl.ds(base, _L)] = mn
            redmax_v[pl.ds(base, _L)] = mx
            return 0

        jax.lax.fori_loop(0, KCH, lr_body, 0)
        pltpu.sync_copy(redmin_v, sh_min.at[lb, sid])
        pltpu.sync_copy(redmax_v, sh_max.at[lb, sid])
        cnt_st[...] = cnt
        pltpu.sync_copy(cnt_st, sh_cnt.at[lb, sid])

    plsc.subcore_barrier()

    @pl.when(sid < B2)
    def _():
        lb = sid
        b = cid * B2 + lb
        pltpu.sync_copy(sh_min.at[lb], gmin_v)
        pltpu.sync_copy(sh_max.at[lb], gmax_v)
        pltpu.sync_copy(sh_cnt.at[lb], gcnt_v)

        def comb_body(k, _):
            base = k * _L
            mn = gmin_v[0, pl.ds(base, _L)]
            mx = gmax_v[0, pl.ds(base, _L)]
            for tt in range(1, _NT):
                mn = jnp.minimum(mn, gmin_v[tt, pl.ds(base, _L)])
                mx = jnp.maximum(mx, gmax_v[tt, pl.ds(base, _L)])
            redmin_v[pl.ds(base, _L)] = mn
            redmax_v[pl.ds(base, _L)] = mx
            return 0

        jax.lax.fori_loop(0, KCH, comb_body, 0)

        sa_v[pl.ds(E, _L)] = jnp.full((_L,), INF, jnp.float32)
        carry = -INF
        for k in range(KCH - 1, -1, -1):
            v = redmin_v[pl.ds(k * _L, _L)]
            rnm = jax.lax.rev(-v, (0,))
            sm = jnp.maximum(plsc.cummax(rnm), carry)
            sa_v[pl.ds(k * _L, _L)] = jax.lax.rev(-sm, (0,))
            carry = jnp.max(sm)

        carry2 = -INF
        for k in range(KCH):
            v = redmax_v[pl.ds(k * _L, _L)]
            pm = jnp.maximum(plsc.cummax(v), carry2)
            pb_v[pl.ds(k * _L, _L)] = pm
            carry2 = jnp.max(pm)

        s0 = jnp.zeros((_L,), jnp.float32)
        s1 = jnp.zeros((_L,), jnp.float32)
        for k in range(KCH):
            scv = plsc.load_gather(sortc_v, [lane + (lb * E + k * _L)])
            sa1 = plsc.load_gather(sa_v, [lane + (k * _L + 1)])
            pb = pb_v[pl.ds(k * _L, _L)]
            res = jnp.minimum(sa1 - scv, scv - pb)
            resp = jnp.minimum(res, jnp.abs(scv))
            real = (k * _L + lane) <= (E - 2)
            s0 = s0 + jnp.where(real, res, 0.0)
            s1 = s1 + jnp.where(real, resp, 0.0)

        cacc = gcnt_v[0]
        for tt in range(1, _NT):
            cacc = cacc + gcnt_v[tt]

        t0 = jnp.sum(s0)
        t1 = jnp.sum(s1)
        tc = jnp.sum(cacc)
        outv = jnp.where(lane == 0, t0,
                         jnp.where(lane == 1, t1,
                                   jnp.where(lane == 2, tc, 0.0)))
        out_st[...] = outv
        pltpu.sync_copy(out_st, out_hbm.at[pl.ds(b * _L, _L)])


def _scratch_types(B, E, T):
    f32 = jnp.float32
    i32 = jnp.int32
    TPT = T // _NT
    return [
        pltpu.VMEM((E + 128,), f32),
        pltpu.VMEM((E,), f32),
        pltpu.VMEM(((B // _NC) * E,), f32),
        pltpu.VMEM((_L * E,), f32),
        pltpu.VMEM((_L * E,), f32),
        pltpu.VMEM((E,), f32),
        pltpu.VMEM((E,), f32),
        pltpu.VMEM((TPT,), f32),
        pltpu.VMEM((E + 128,), f32),
        pltpu.VMEM((E,), f32),
        pltpu.VMEM((_NT, E), f32),
        pltpu.VMEM((_NT, E), f32),
        pltpu.VMEM((_NT, _L), f32),
        pltpu.VMEM((_L,), i32),
        pltpu.VMEM((_L,), f32),
        pltpu.VMEM((_L,), f32),
        pltpu.VMEM((_NT, _L), i32),
        pltpu.VMEM((_NT, _L), f32),
        pltpu.VMEM((_L,), f32),
        pltpu.VMEM_SHARED((B // _NC, _NT, _L), i32),
        pltpu.VMEM_SHARED((B // _NC, _NT, _L), f32),
        pltpu.VMEM_SHARED((B // _NC, _NT, E), f32),
        pltpu.VMEM_SHARED((B // _NC, _NT, E), f32),
        pltpu.VMEM_SHARED((B // _NC, _NT, _L), f32),
    ]


def _make_call(B, E, T):
    mesh = plsc.VectorSubcoreMesh(core_axis_name="c", subcore_axis_name="s",
                                  num_cores=_NC, num_subcores=_NT)
    return functools.partial(
        pl.kernel, mesh=mesh,
        out_type=jax.ShapeDtypeStruct((B * _L,), jnp.float32),
        scratch_types=_scratch_types(B, E, T),
        compiler_params=pltpu.CompilerParams(use_tc_tiling_on_sc=False,
                                             needs_layout_passes=False),
    )(_sc_body)


_sc_call = _make_call(4, 256, 50176)


def kernel(bins, target_depth_maps):
    B = bins.shape[0]
    target = target_depth_maps.reshape(-1).astype(jnp.float32)
    out = _sc_call(bins.reshape(-1).astype(jnp.float32), target).reshape(B, _L)
    s0, s1, cnt = out[:, 0], out[:, 1], out[:, 2]
    max_len = jnp.max(cnt)
    return jnp.sum(jnp.where(cnt < max_len, s1, s0))
